# baseline (speedup 1.0000x reference)
import functools

import jax
import jax.numpy as jnp
from jax import lax
from jax.experimental import pallas as pl
from jax.experimental.pallas import tpu as pltpu
from jax.experimental.pallas import tpu_sc as plsc

N = 10000
NP = 10240
E = 320000
D = 128
CH = 128
NCHUNK = E // CH
NC, NS = 2, 16
NW = NC * NS
JMAX = -(-NCHUNK // NW)
RPT = NP // NS
ZB = 128

_mesh = plsc.VectorSubcoreMesh(core_axis_name="c", subcore_axis_name="s")


def _fill_rows_d(buf, val):
    def body(i, _):
        for j in range(D // 16):
            buf[i, pl.ds(j * 16, 16)] = jnp.full((16,), val, jnp.float32)
        return 0
    lax.fori_loop(0, buf.shape[0], body, 0)


def _zero_rows_d(buf):
    _fill_rows_d(buf, 0.0)


def _spmem_to_hbm(sh, out, core, tile, stage):
    rows = stage.shape[0]
    for r in range(RPT // rows):
        sl = pl.ds(tile * RPT + r * rows, rows)
        pltpu.sync_copy(sh.at[sl], stage)
        pltpu.sync_copy(stage, out.at[core, sl])


def _pipe_accum(wid, acc_sh, fetch_start, fetch_wait, ix, rows, sems):
    def drain(b):
        fetch_wait(b)
        pltpu.sync_copy(rows[b], acc_sh.at[ix[b].at[1]], add=True)

    def guarded(cid, fn):
        @pl.when(cid < NCHUNK)
        def _():
            fn()

    fetch_start(0, wid)
    niter = (JMAX + 1) // 2

    def body(i, _):
        j0 = 2 * i
        cid1 = wid + NW * (j0 + 1)
        cid0 = wid + NW * j0
        cid2 = wid + NW * (j0 + 2)
        guarded(cid1, lambda: fetch_start(1, cid1))
        guarded(cid0, lambda: drain(0))
        guarded(cid2, lambda: fetch_start(0, cid2))
        guarded(cid1, lambda: drain(1))
        return 0

    lax.fori_loop(0, niter, body, 0)


@functools.partial(
    pl.kernel,
    out_type=jax.ShapeDtypeStruct((NC, NP, D), jnp.float32),
    mesh=_mesh,
    scratch_types=[
        pltpu.VMEM_SHARED((NP, D), jnp.float32),
        pltpu.VMEM((2, CH), jnp.int32),
        pltpu.VMEM((2, CH), jnp.int32),
        pltpu.VMEM((CH, D), jnp.float32),
        pltpu.VMEM((CH, D), jnp.float32),
        pltpu.SemaphoreType.DMA,
        pltpu.SemaphoreType.DMA,
    ],
)
def _sc_edge_accum(ea_hbm, eidx_hbm, sea_out,
                   sea_sh, ix0, ix1, rows0, rows1, sem0, sem1):
    c = lax.axis_index("c")
    s = lax.axis_index("s")
    wid = s * NC + c
    ix = (ix0, ix1)
    rows = (rows0, rows1)
    sems = (sem0, sem1)
    _zero_rows_d(rows0)
    for r in range(RPT // CH):
        pltpu.sync_copy(rows0, sea_sh.at[pl.ds(s * RPT + r * CH, CH)])
    plsc.subcore_barrier()

    def fetch_start(b, cid):
        pltpu.sync_copy(eidx_hbm.at[cid], ix[b])
        pltpu.async_copy(ea_hbm.at[pl.ds(cid * CH, CH)], rows[b], sems[b])

    def fetch_wait(b):
        pltpu.make_async_copy(ea_hbm.at[pl.ds(0, CH)], rows[b], sems[b]).wait()

    _pipe_accum(wid, sea_sh, fetch_start, fetch_wait, ix, rows, sems)
    plsc.subcore_barrier()
    _spmem_to_hbm(sea_sh, sea_out, c, s, rows0)


@functools.partial(
    pl.kernel,
    out_type=jax.ShapeDtypeStruct((NC, NP, D), jnp.float32),
    mesh=_mesh,
    scratch_types=[
        pltpu.VMEM_SHARED((NP, D), jnp.float32),
        pltpu.VMEM((2, CH), jnp.int32),
        pltpu.VMEM((2, CH), jnp.int32),
        pltpu.VMEM((CH, D), jnp.float32),
        pltpu.SemaphoreType.DMA,
        pltpu.SemaphoreType.DMA,
    ],
)
def _sc_deg(eidx_hbm, cnt_out, cnt_sh, ix0, ix1, ones_v, sem0, sem1):
    c = lax.axis_index("c")
    s = lax.axis_index("s")
    wid = s * NC + c
    ix = (ix0, ix1)
    sems = (sem0, sem1)
    _zero_rows_d(ones_v)
    for r in range(RPT // CH):
        pltpu.sync_copy(ones_v, cnt_sh.at[pl.ds(s * RPT + r * CH, CH)])
    plsc.subcore_barrier()
    _fill_rows_d(ones_v, 1.0)

    def fetch_start(b, cid):
        pltpu.async_copy(eidx_hbm.at[cid], ix[b], sems[b])

    def fetch_wait(b):
        pltpu.make_async_copy(eidx_hbm.at[0], ix[b], sems[b]).wait()

    _pipe_accum(wid, cnt_sh, fetch_start, fetch_wait, ix,
                (ones_v, ones_v), sems)
    plsc.subcore_barrier()
    _spmem_to_hbm(cnt_sh, cnt_out, c, s, ones_v)


NBLK = 20
NBLKD = 21
EPADW = NBLKD * 4 * CH


@functools.partial(
    pl.kernel,
    out_type=jax.ShapeDtypeStruct((NC, NP, D), jnp.float32),
    mesh=_mesh,
    scratch_types=[
        pltpu.VMEM_SHARED((NP, D), jnp.float32),
        pltpu.VMEM((8, CH), jnp.int32),
        pltpu.VMEM((8, CH), jnp.int32),
        pltpu.VMEM((CH, D), jnp.float32),
        pltpu.VMEM((CH, D), jnp.float32),
        pltpu.SemaphoreType.DMA,
        pltpu.SemaphoreType.DMA,
    ],
)
def _sc_spmm(h_hbm, eb_hbm, p_out,
             p_sh, jxa, jxb, r0, r1, sem0, sem1):
    c = lax.axis_index("c")
    s = lax.axis_index("s")
    wid = s * NC + c
    _zero_rows_d(r0)
    for r in range(RPT // CH):
        pltpu.sync_copy(r0, p_sh.at[pl.ds(s * RPT + r * CH, CH)])
    plsc.subcore_barrier()

    def g(rbuf, jx, m, sem):
        pltpu.async_copy(h_hbm.at[jx.at[2 * m]], rbuf, sem)

    def gw(rbuf, sem):
        pltpu.make_async_copy(h_hbm.at[jxa.at[0]], rbuf, sem).wait()

    def sc(rbuf, jx, m):
        pltpu.sync_copy(rbuf, p_sh.at[jx.at[2 * m + 1]], add=True)

    pltpu.sync_copy(eb_hbm.at[wid, 0], jxa)
    g(r0, jxa, 0, sem0)

    def body(i, _):
        pltpu.sync_copy(eb_hbm.at[wid, 2 * i + 1], jxb)
        g(r1, jxa, 1, sem1)
        gw(r0, sem0); sc(r0, jxa, 0)
        g(r0, jxa, 2, sem0)
        gw(r1, sem1); sc(r1, jxa, 1)
        g(r1, jxa, 3, sem1)
        gw(r0, sem0); sc(r0, jxa, 2)
        g(r0, jxb, 0, sem0)
        gw(r1, sem1); sc(r1, jxa, 3)
        pltpu.sync_copy(eb_hbm.at[wid, 2 * i + 2], jxa)
        g(r1, jxb, 1, sem1)
        gw(r0, sem0); sc(r0, jxb, 0)
        g(r0, jxb, 2, sem0)
        gw(r1, sem1); sc(r1, jxb, 1)
        g(r1, jxb, 3, sem1)
        gw(r0, sem0); sc(r0, jxb, 2)
        g(r0, jxa, 0, sem0)
        gw(r1, sem1); sc(r1, jxb, 3)
        return 0

    lax.fori_loop(0, NBLK // 2, body, 0)
    gw(r0, sem0)
    plsc.subcore_barrier()
    _spmem_to_hbm(p_sh, p_out, c, s, r0)


def _mm_relu_body(x_ref, w_ref, b_ref, o_ref):
    o_ref[...] = jnp.maximum(
        jnp.dot(x_ref[...], w_ref[...], preferred_element_type=jnp.float32)
        + b_ref[...], 0.0)


def _emb(x, w, b, bm):
    m, k = x.shape
    n = w.shape[1]
    return pl.pallas_call(
        _mm_relu_body,
        grid=(m // bm,),
        in_specs=[pl.BlockSpec((bm, k), lambda i: (i, 0)),
                  pl.BlockSpec((k, n), lambda i: (0, 0)),
                  pl.BlockSpec((n,), lambda i: (0,))],
        out_specs=pl.BlockSpec((bm, n), lambda i: (i, 0)),
        out_shape=jax.ShapeDtypeStruct((m, n), jnp.float32),
    )(x, w, b)


def _update_h(k, h_ref, p0_ref, p1_ref, se0_ref, se1_ref, c0_ref, c1_ref,
              wt_ref, wb_ref, cb_ref, ut_ref, ub_ref, ubias_ref):
    P = p0_ref[0] + p1_ref[0]
    S = se0_ref[0] + se1_ref[0]
    cnt = c0_ref[0, :, 0:1] + c1_ref[0, :, 0:1]
    inv = 1.0 / jnp.maximum(cnt, 1.0)
    gate = cnt * inv
    numer = (jnp.dot(P, wt_ref[0], preferred_element_type=jnp.float32)
             + jnp.dot(S, wb_ref[0], preferred_element_type=jnp.float32))
    aggr = numer * inv + gate * cb_ref[k]
    return jnp.maximum(
        jnp.dot(h_ref[...], ut_ref[0], preferred_element_type=jnp.float32)
        + jnp.dot(aggr, ub_ref[0], preferred_element_type=jnp.float32)
        + ubias_ref[k], 0.0)


def _update(h, p_all, sea_all, cnt_all, k, cew, ceb, cuw, cub,
            dec=None, bm=1000):
    row = lambda i: (i, 0)
    full = lambda i: (0, 0)
    part0 = lambda i: (0, i, 0)
    part1 = lambda i: (1, i, 0)
    nk = ceb.shape[0]
    wtop = lambda i: (k, 0, 0)
    wbot = lambda i: (k, 1, 0)
    in_specs = [pl.BlockSpec((bm, D), row),
                pl.BlockSpec((1, bm, D), part0),
                pl.BlockSpec((1, bm, D), part1),
                pl.BlockSpec((1, bm, D), part0),
                pl.BlockSpec((1, bm, D), part1),
                pl.BlockSpec((1, bm, D), part0),
                pl.BlockSpec((1, bm, D), part1),
                pl.BlockSpec((1, D, D), wtop), pl.BlockSpec((1, D, D), wbot),
                pl.BlockSpec((nk, D), full),
                pl.BlockSpec((1, D, D), wtop), pl.BlockSpec((1, D, D), wbot),
                pl.BlockSpec((nk, D), full)]
    args = [h, p_all, p_all, sea_all, sea_all, cnt_all, cnt_all,
            cew, cew, ceb, cuw, cuw, cub]
    if dec is None:
        def body(*refs):
            refs[-1][...] = _update_h(k, *refs[:-1])
        return pl.pallas_call(
            body,
            grid=(N // bm,),
            in_specs=in_specs,
            out_specs=pl.BlockSpec((bm, D), row),
            out_shape=jax.ShapeDtypeStruct((N, D), jnp.float32),
        )(*args)
    dw, db = dec
    n_out = dw.shape[1]

    def body_dec(*refs):
        hv = _update_h(k, *refs[:-3])
        refs[-1][...] = (jnp.dot(hv, refs[-3][...],
                                 preferred_element_type=jnp.float32)
                         + refs[-2][...])
    return pl.pallas_call(
        body_dec,
        grid=(N // bm,),
        in_specs=in_specs + [pl.BlockSpec((D, n_out), full),
                             pl.BlockSpec((n_out,), lambda i: (0,))],
        out_specs=pl.BlockSpec((bm, n_out), row),
        out_shape=jax.ShapeDtypeStruct((N, n_out), jnp.float32),
    )(*args, dw, db)


def _decode_body(h_ref, w_ref, b_ref, o_ref):
    o_ref[...] = (jnp.dot(h_ref[...], w_ref[...],
                          preferred_element_type=jnp.float32) + b_ref[...])


def _decode(h, w, b, bm=1000):
    n_out = w.shape[1]
    return pl.pallas_call(
        _decode_body,
        grid=(N // bm,),
        in_specs=[pl.BlockSpec((bm, D), lambda i: (i, 0)),
                  pl.BlockSpec((D, n_out), lambda i: (0, 0)),
                  pl.BlockSpec((1, n_out), lambda i: (0, 0))],
        out_specs=pl.BlockSpec((bm, n_out), lambda i: (i, 0)),
        out_shape=jax.ShapeDtypeStruct((N, n_out), jnp.float32),
    )(h, w, b)


def kernel(x, edge_attr, edge_index, node_W, node_b, edge_W, edge_b,
           conv_edge_W, conv_edge_b, conv_upd_W, conv_upd_b, dec_W, dec_b):
    eidx = edge_index.reshape(2, NCHUNK, CH).transpose(1, 0, 2)
    padn = NW * EPADW - E
    src_p = jnp.pad(edge_index[0], (0, padn))
    dst_p = jnp.pad(edge_index[1], (0, padn), constant_values=NP - 1)
    eidx_b = (jnp.stack([src_p, dst_p])
              .reshape(2, NBLKD * 4, NW, CH)
              .transpose(2, 1, 0, 3)
              .reshape(NW, NBLKD, 8, CH))
    h = _emb(x, node_W, node_b, bm=1000)
    ea = _emb(edge_attr, edge_W, edge_b, bm=4000)
    sea_all = _sc_edge_accum(ea, eidx)
    cnt_all = _sc_deg(eidx)
    nk = conv_edge_W.shape[0]
    for k in range(nk):
        p_all = _sc_spmm(h, eidx_b)
        dec = (dec_W, dec_b) if k == nk - 1 else None
        h = _update(h, p_all, sea_all, cnt_all, k,
                    conv_edge_W, conv_edge_b, conv_upd_W, conv_upd_b,
                    dec=dec)
    return h


# speedup vs baseline: 2.5456x; 2.5456x over previous
import functools

import jax
import jax.numpy as jnp
from jax import lax
from jax.experimental import pallas as pl
from jax.experimental.pallas import tpu as pltpu
from jax.experimental.pallas import tpu_sc as plsc

N = 10000
NP = 10240
E = 320000
D = 128
CH = 128
NCHUNK = E // CH
NC, NS = 2, 16
NW = NC * NS
JMAX = -(-NCHUNK // NW)
RPT = NP // NS
ZB = 128

_mesh = plsc.VectorSubcoreMesh(core_axis_name="c", subcore_axis_name="s")


def _fill_rows_d(buf, val):
    def body(i, _):
        for j in range(D // 16):
            buf[i, pl.ds(j * 16, 16)] = jnp.full((16,), val, jnp.float32)
        return 0
    lax.fori_loop(0, buf.shape[0], body, 0)


def _zero_rows_d(buf):
    _fill_rows_d(buf, 0.0)


def _spmem_to_hbm(sh, out, core, tile, stage):
    rows = stage.shape[0]
    for r in range(RPT // rows):
        sl = pl.ds(tile * RPT + r * rows, rows)
        pltpu.sync_copy(sh.at[sl], stage)
        pltpu.sync_copy(stage, out.at[core, sl])


def _pipe_accum(wid, acc_sh, fetch_start, fetch_wait, ix, rows, sems):
    def drain(b):
        fetch_wait(b)
        pltpu.sync_copy(rows[b], acc_sh.at[ix[b].at[1]], add=True)

    def guarded(cid, fn):
        @pl.when(cid < NCHUNK)
        def _():
            fn()

    fetch_start(0, wid)
    niter = (JMAX + 1) // 2

    def body(i, _):
        j0 = 2 * i
        cid1 = wid + NW * (j0 + 1)
        cid0 = wid + NW * j0
        cid2 = wid + NW * (j0 + 2)
        guarded(cid1, lambda: fetch_start(1, cid1))
        guarded(cid0, lambda: drain(0))
        guarded(cid2, lambda: fetch_start(0, cid2))
        guarded(cid1, lambda: drain(1))
        return 0

    lax.fori_loop(0, niter, body, 0)


@functools.partial(
    pl.kernel,
    out_type=jax.ShapeDtypeStruct((NC, NP, D), jnp.float32),
    mesh=_mesh,
    scratch_types=[
        pltpu.VMEM_SHARED((NP, D), jnp.float32),
        pltpu.VMEM((2, CH), jnp.int32),
        pltpu.VMEM((2, CH), jnp.int32),
        pltpu.VMEM((CH, D), jnp.float32),
        pltpu.VMEM((CH, D), jnp.float32),
        pltpu.SemaphoreType.DMA,
        pltpu.SemaphoreType.DMA,
    ],
)
def _sc_edge_accum(ea_hbm, eidx_hbm, sea_out,
                   sea_sh, ix0, ix1, rows0, rows1, sem0, sem1):
    c = lax.axis_index("c")
    s = lax.axis_index("s")
    wid = s * NC + c
    ix = (ix0, ix1)
    rows = (rows0, rows1)
    sems = (sem0, sem1)
    _zero_rows_d(rows0)
    for r in range(RPT // CH):
        pltpu.sync_copy(rows0, sea_sh.at[pl.ds(s * RPT + r * CH, CH)])
    plsc.subcore_barrier()

    def fetch_start(b, cid):
        pltpu.sync_copy(eidx_hbm.at[cid], ix[b])
        pltpu.async_copy(ea_hbm.at[pl.ds(cid * CH, CH)], rows[b], sems[b])

    def fetch_wait(b):
        pltpu.make_async_copy(ea_hbm.at[pl.ds(0, CH)], rows[b], sems[b]).wait()

    _pipe_accum(wid, sea_sh, fetch_start, fetch_wait, ix, rows, sems)
    plsc.subcore_barrier()
    _spmem_to_hbm(sea_sh, sea_out, c, s, rows0)


@functools.partial(
    pl.kernel,
    out_type=jax.ShapeDtypeStruct((NC, NP, D), jnp.float32),
    mesh=_mesh,
    scratch_types=[
        pltpu.VMEM_SHARED((NP, D), jnp.float32),
        pltpu.VMEM((2, CH), jnp.int32),
        pltpu.VMEM((2, CH), jnp.int32),
        pltpu.VMEM((CH, D), jnp.float32),
        pltpu.SemaphoreType.DMA,
        pltpu.SemaphoreType.DMA,
    ],
)
def _sc_deg(eidx_hbm, cnt_out, cnt_sh, ix0, ix1, ones_v, sem0, sem1):
    c = lax.axis_index("c")
    s = lax.axis_index("s")
    wid = s * NC + c
    ix = (ix0, ix1)
    sems = (sem0, sem1)
    _zero_rows_d(ones_v)
    for r in range(RPT // CH):
        pltpu.sync_copy(ones_v, cnt_sh.at[pl.ds(s * RPT + r * CH, CH)])
    plsc.subcore_barrier()
    _fill_rows_d(ones_v, 1.0)

    def fetch_start(b, cid):
        pltpu.async_copy(eidx_hbm.at[cid], ix[b], sems[b])

    def fetch_wait(b):
        pltpu.make_async_copy(eidx_hbm.at[0], ix[b], sems[b]).wait()

    _pipe_accum(wid, cnt_sh, fetch_start, fetch_wait, ix,
                (ones_v, ones_v), sems)
    plsc.subcore_barrier()
    _spmem_to_hbm(cnt_sh, cnt_out, c, s, ones_v)


@functools.partial(
    pl.kernel,
    out_type=jax.ShapeDtypeStruct((NC, NP, D), jnp.float32),
    mesh=_mesh,
    scratch_types=[
        pltpu.VMEM_SHARED((NP, D), jnp.float32),
        pltpu.VMEM((2, CH), jnp.int32),
        pltpu.VMEM((2, CH), jnp.int32),
        pltpu.VMEM((CH, D), jnp.float32),
        pltpu.VMEM((CH, D), jnp.float32),
        pltpu.SemaphoreType.DMA,
        pltpu.SemaphoreType.DMA,
    ],
)
def _sc_spmm(h_hbm, eidx_hbm, p_out,
             p_sh, ix0, ix1, rows0, rows1, sem0, sem1):
    c = lax.axis_index("c")
    s = lax.axis_index("s")
    wid = s * NC + c
    ix = (ix0, ix1)
    rows = (rows0, rows1)
    sems = (sem0, sem1)
    _zero_rows_d(rows0)
    for r in range(RPT // CH):
        pltpu.sync_copy(rows0, p_sh.at[pl.ds(s * RPT + r * CH, CH)])
    plsc.subcore_barrier()

    def fetch_start(b, cid):
        pltpu.sync_copy(eidx_hbm.at[cid], ix[b])
        pltpu.async_copy(h_hbm.at[ix[b].at[0]], rows[b], sems[b])

    def fetch_wait(b):
        pltpu.make_async_copy(h_hbm.at[ix[b].at[0]], rows[b], sems[b]).wait()

    _pipe_accum(wid, p_sh, fetch_start, fetch_wait, ix, rows, sems)
    plsc.subcore_barrier()
    _spmem_to_hbm(p_sh, p_out, c, s, rows0)


def _mm_relu_body(x_ref, w_ref, b_ref, o_ref):
    o_ref[...] = jnp.maximum(
        jnp.dot(x_ref[...], w_ref[...], preferred_element_type=jnp.float32)
        + b_ref[...], 0.0)


def _emb(x, w, b, bm):
    m, k = x.shape
    n = w.shape[1]
    return pl.pallas_call(
        _mm_relu_body,
        grid=(m // bm,),
        in_specs=[pl.BlockSpec((bm, k), lambda i: (i, 0)),
                  pl.BlockSpec((k, n), lambda i: (0, 0)),
                  pl.BlockSpec((n,), lambda i: (0,))],
        out_specs=pl.BlockSpec((bm, n), lambda i: (i, 0)),
        out_shape=jax.ShapeDtypeStruct((m, n), jnp.float32),
    )(x, w, b)


def _update_h(k, h_ref, p0_ref, p1_ref, se0_ref, se1_ref, c0_ref, c1_ref,
              wt_ref, wb_ref, cb_ref, ut_ref, ub_ref, ubias_ref):
    P = p0_ref[0] + p1_ref[0]
    S = se0_ref[0] + se1_ref[0]
    cnt = c0_ref[0, :, 0:1] + c1_ref[0, :, 0:1]
    inv = 1.0 / jnp.maximum(cnt, 1.0)
    gate = cnt * inv
    numer = (jnp.dot(P, wt_ref[0], preferred_element_type=jnp.float32)
             + jnp.dot(S, wb_ref[0], preferred_element_type=jnp.float32))
    aggr = numer * inv + gate * cb_ref[k]
    return jnp.maximum(
        jnp.dot(h_ref[...], ut_ref[0], preferred_element_type=jnp.float32)
        + jnp.dot(aggr, ub_ref[0], preferred_element_type=jnp.float32)
        + ubias_ref[k], 0.0)


def _update(h, p_all, sea_all, cnt_all, k, cew, ceb, cuw, cub,
            dec=None, bm=1000):
    row = lambda i: (i, 0)
    full = lambda i: (0, 0)
    part0 = lambda i: (0, i, 0)
    part1 = lambda i: (1, i, 0)
    nk = ceb.shape[0]
    wtop = lambda i: (k, 0, 0)
    wbot = lambda i: (k, 1, 0)
    in_specs = [pl.BlockSpec((bm, D), row),
                pl.BlockSpec((1, bm, D), part0),
                pl.BlockSpec((1, bm, D), part1),
                pl.BlockSpec((1, bm, D), part0),
                pl.BlockSpec((1, bm, D), part1),
                pl.BlockSpec((1, bm, D), part0),
                pl.BlockSpec((1, bm, D), part1),
                pl.BlockSpec((1, D, D), wtop), pl.BlockSpec((1, D, D), wbot),
                pl.BlockSpec((nk, D), full),
                pl.BlockSpec((1, D, D), wtop), pl.BlockSpec((1, D, D), wbot),
                pl.BlockSpec((nk, D), full)]
    args = [h, p_all, p_all, sea_all, sea_all, cnt_all, cnt_all,
            cew, cew, ceb, cuw, cuw, cub]
    if dec is None:
        def body(*refs):
            refs[-1][...] = _update_h(k, *refs[:-1])
        return pl.pallas_call(
            body,
            grid=(N // bm,),
            in_specs=in_specs,
            out_specs=pl.BlockSpec((bm, D), row),
            out_shape=jax.ShapeDtypeStruct((N, D), jnp.float32),
        )(*args)
    dw, db = dec
    n_out = dw.shape[1]

    def body_dec(*refs):
        hv = _update_h(k, *refs[:-3])
        refs[-1][...] = (jnp.dot(hv, refs[-3][...],
                                 preferred_element_type=jnp.float32)
                         + refs[-2][...])
    return pl.pallas_call(
        body_dec,
        grid=(N // bm,),
        in_specs=in_specs + [pl.BlockSpec((D, n_out), full),
                             pl.BlockSpec((n_out,), lambda i: (0,))],
        out_specs=pl.BlockSpec((bm, n_out), row),
        out_shape=jax.ShapeDtypeStruct((N, n_out), jnp.float32),
    )(*args, dw, db)


def _decode_body(h_ref, w_ref, b_ref, o_ref):
    o_ref[...] = (jnp.dot(h_ref[...], w_ref[...],
                          preferred_element_type=jnp.float32) + b_ref[...])


def _decode(h, w, b, bm=1000):
    n_out = w.shape[1]
    return pl.pallas_call(
        _decode_body,
        grid=(N // bm,),
        in_specs=[pl.BlockSpec((bm, D), lambda i: (i, 0)),
                  pl.BlockSpec((D, n_out), lambda i: (0, 0)),
                  pl.BlockSpec((1, n_out), lambda i: (0, 0))],
        out_specs=pl.BlockSpec((bm, n_out), lambda i: (i, 0)),
        out_shape=jax.ShapeDtypeStruct((N, n_out), jnp.float32),
    )(h, w, b)


def kernel(x, edge_attr, edge_index, node_W, node_b, edge_W, edge_b,
           conv_edge_W, conv_edge_b, conv_upd_W, conv_upd_b, dec_W, dec_b):
    eidx = edge_index.reshape(2, NCHUNK, CH).transpose(1, 0, 2)
    h = _emb(x, node_W, node_b, bm=1000)
    ea = _emb(edge_attr, edge_W, edge_b, bm=4000)
    sea_all = _sc_edge_accum(ea, eidx)
    cnt_all = _sc_deg(eidx)
    nk = conv_edge_W.shape[0]
    for k in range(nk):
        p_all = _sc_spmm(h, eidx)
        dec = (dec_W, dec_b) if k == nk - 1 else None
        h = _update(h, p_all, sea_all, cnt_all, k,
                    conv_edge_W, conv_edge_b, conv_upd_W, conv_upd_b,
                    dec=dec)
    return h


# speedup vs baseline: 2.5623x; 1.0066x over previous
import functools

import jax
import jax.numpy as jnp
from jax import lax
from jax.experimental import pallas as pl
from jax.experimental.pallas import tpu as pltpu
from jax.experimental.pallas import tpu_sc as plsc

N = 10000
NP = 10240
E = 320000
D = 128
CH = 128
NCHUNK = E // CH
NC, NS = 2, 16
NW = NC * NS
JMAX = -(-NCHUNK // NW)
RPT = NP // NS
ZB = 128

_mesh = plsc.VectorSubcoreMesh(core_axis_name="c", subcore_axis_name="s")


def _fill_rows_d(buf, val):
    def body(i, _):
        for j in range(D // 16):
            buf[i, pl.ds(j * 16, 16)] = jnp.full((16,), val, jnp.float32)
        return 0
    lax.fori_loop(0, buf.shape[0], body, 0)


def _zero_rows_d(buf):
    _fill_rows_d(buf, 0.0)


def _spmem_to_hbm(sh, out, core, tile, stage):
    rows = stage.shape[0]
    for r in range(RPT // rows):
        sl = pl.ds(tile * RPT + r * rows, rows)
        pltpu.sync_copy(sh.at[sl], stage)
        pltpu.sync_copy(stage, out.at[core, sl])


def _pipe_accum(wid, acc_sh, fetch_start, fetch_wait, ix, rows, sems):
    def drain(b):
        fetch_wait(b)
        pltpu.sync_copy(rows[b], acc_sh.at[ix[b].at[1]], add=True)

    def guarded(cid, fn):
        @pl.when(cid < NCHUNK)
        def _():
            fn()

    fetch_start(0, wid)
    niter = (JMAX + 1) // 2

    def body(i, _):
        j0 = 2 * i
        cid1 = wid + NW * (j0 + 1)
        cid0 = wid + NW * j0
        cid2 = wid + NW * (j0 + 2)
        guarded(cid1, lambda: fetch_start(1, cid1))
        guarded(cid0, lambda: drain(0))
        guarded(cid2, lambda: fetch_start(0, cid2))
        guarded(cid1, lambda: drain(1))
        return 0

    lax.fori_loop(0, niter, body, 0)


@functools.partial(
    pl.kernel,
    out_type=[jax.ShapeDtypeStruct((NC, NP, D), jnp.float32),
              jax.ShapeDtypeStruct((NC, NP, D), jnp.float32)],
    mesh=_mesh,
    scratch_types=[
        pltpu.VMEM_SHARED((NP, D), jnp.float32),
        pltpu.VMEM((2, CH), jnp.int32),
        pltpu.VMEM((2, CH), jnp.int32),
        pltpu.VMEM((CH, D), jnp.float32),
        pltpu.VMEM((CH, D), jnp.float32),
        pltpu.SemaphoreType.DMA,
        pltpu.SemaphoreType.DMA,
    ],
)
def _sc_edge_accum(ea_hbm, eidx_hbm, sea_out, cnt_out,
                   sea_sh, ix0, ix1, rows0, rows1, sem0, sem1):
    c = lax.axis_index("c")
    s = lax.axis_index("s")
    wid = s * NC + c
    ix = (ix0, ix1)
    rows = (rows0, rows1)
    sems = (sem0, sem1)
    _zero_rows_d(rows0)
    for r in range(RPT // CH):
        pltpu.sync_copy(rows0, sea_sh.at[pl.ds(s * RPT + r * CH, CH)])
    plsc.subcore_barrier()

    def fetch_start(b, cid):
        pltpu.sync_copy(eidx_hbm.at[cid], ix[b])
        pltpu.async_copy(ea_hbm.at[pl.ds(cid * CH, CH)], rows[b], sems[b])

    def fetch_wait(b):
        pltpu.make_async_copy(ea_hbm.at[pl.ds(0, CH)], rows[b], sems[b]).wait()

    _pipe_accum(wid, sea_sh, fetch_start, fetch_wait, ix, rows, sems)
    plsc.subcore_barrier()
    _spmem_to_hbm(sea_sh, sea_out, c, s, rows0)
    plsc.subcore_barrier()

    _zero_rows_d(rows1)
    for r in range(RPT // CH):
        pltpu.sync_copy(rows1, sea_sh.at[pl.ds(s * RPT + r * CH, CH)])
    plsc.subcore_barrier()
    _fill_rows_d(rows0, 1.0)

    def deg_start(b, cid):
        pltpu.async_copy(eidx_hbm.at[cid], ix[b], sems[b])

    def deg_wait(b):
        pltpu.make_async_copy(eidx_hbm.at[0], ix[b], sems[b]).wait()

    _pipe_accum(wid, sea_sh, deg_start, deg_wait, ix, (rows0, rows0), sems)
    plsc.subcore_barrier()
    _spmem_to_hbm(sea_sh, cnt_out, c, s, rows1)


@functools.partial(
    pl.kernel,
    out_type=jax.ShapeDtypeStruct((NC, NP, D), jnp.float32),
    mesh=_mesh,
    scratch_types=[
        pltpu.VMEM_SHARED((NP, D), jnp.float32),
        pltpu.VMEM((2, CH), jnp.int32),
        pltpu.VMEM((2, CH), jnp.int32),
        pltpu.VMEM((CH, D), jnp.float32),
        pltpu.VMEM((CH, D), jnp.float32),
        pltpu.SemaphoreType.DMA,
        pltpu.SemaphoreType.DMA,
    ],
)
def _sc_spmm(h_hbm, eidx_hbm, p_out,
             p_sh, ix0, ix1, rows0, rows1, sem0, sem1):
    c = lax.axis_index("c")
    s = lax.axis_index("s")
    wid = s * NC + c
    ix = (ix0, ix1)
    rows = (rows0, rows1)
    sems = (sem0, sem1)
    _zero_rows_d(rows0)
    for r in range(RPT // CH):
        pltpu.sync_copy(rows0, p_sh.at[pl.ds(s * RPT + r * CH, CH)])
    plsc.subcore_barrier()

    def fetch_start(b, cid):
        pltpu.sync_copy(eidx_hbm.at[cid], ix[b])
        pltpu.async_copy(h_hbm.at[ix[b].at[0]], rows[b], sems[b])

    def fetch_wait(b):
        pltpu.make_async_copy(h_hbm.at[ix[b].at[0]], rows[b], sems[b]).wait()

    _pipe_accum(wid, p_sh, fetch_start, fetch_wait, ix, rows, sems)
    plsc.subcore_barrier()
    _spmem_to_hbm(p_sh, p_out, c, s, rows0)


def _mm_relu_body(x_ref, w_ref, b_ref, o_ref):
    o_ref[...] = jnp.maximum(
        jnp.dot(x_ref[...], w_ref[...], preferred_element_type=jnp.float32)
        + b_ref[...], 0.0)


def _emb(x, w, b, bm):
    m, k = x.shape
    n = w.shape[1]
    return pl.pallas_call(
        _mm_relu_body,
        grid=(m // bm,),
        in_specs=[pl.BlockSpec((bm, k), lambda i: (i, 0)),
                  pl.BlockSpec((k, n), lambda i: (0, 0)),
                  pl.BlockSpec((n,), lambda i: (0,))],
        out_specs=pl.BlockSpec((bm, n), lambda i: (i, 0)),
        out_shape=jax.ShapeDtypeStruct((m, n), jnp.float32),
    )(x, w, b)


def _update_h(k, h_ref, p0_ref, p1_ref, se0_ref, se1_ref, c0_ref, c1_ref,
              wt_ref, wb_ref, cb_ref, ut_ref, ub_ref, ubias_ref):
    P = p0_ref[0] + p1_ref[0]
    S = se0_ref[0] + se1_ref[0]
    cnt = c0_ref[0, :, 0:1] + c1_ref[0, :, 0:1]
    inv = 1.0 / jnp.maximum(cnt, 1.0)
    gate = cnt * inv
    numer = (jnp.dot(P, wt_ref[0], preferred_element_type=jnp.float32)
             + jnp.dot(S, wb_ref[0], preferred_element_type=jnp.float32))
    aggr = numer * inv + gate * cb_ref[k]
    return jnp.maximum(
        jnp.dot(h_ref[...], ut_ref[0], preferred_element_type=jnp.float32)
        + jnp.dot(aggr, ub_ref[0], preferred_element_type=jnp.float32)
        + ubias_ref[k], 0.0)


def _update(h, p_all, sea_all, cnt_all, k, cew, ceb, cuw, cub,
            dec=None, bm=1000):
    row = lambda i: (i, 0)
    full = lambda i: (0, 0)
    part0 = lambda i: (0, i, 0)
    part1 = lambda i: (1, i, 0)
    nk = ceb.shape[0]
    wtop = lambda i: (k, 0, 0)
    wbot = lambda i: (k, 1, 0)
    in_specs = [pl.BlockSpec((bm, D), row),
                pl.BlockSpec((1, bm, D), part0),
                pl.BlockSpec((1, bm, D), part1),
                pl.BlockSpec((1, bm, D), part0),
                pl.BlockSpec((1, bm, D), part1),
                pl.BlockSpec((1, bm, D), part0),
                pl.BlockSpec((1, bm, D), part1),
                pl.BlockSpec((1, D, D), wtop), pl.BlockSpec((1, D, D), wbot),
                pl.BlockSpec((nk, D), full),
                pl.BlockSpec((1, D, D), wtop), pl.BlockSpec((1, D, D), wbot),
                pl.BlockSpec((nk, D), full)]
    args = [h, p_all, p_all, sea_all, sea_all, cnt_all, cnt_all,
            cew, cew, ceb, cuw, cuw, cub]
    if dec is None:
        def body(*refs):
            refs[-1][...] = _update_h(k, *refs[:-1])
        return pl.pallas_call(
            body,
            grid=(N // bm,),
            in_specs=in_specs,
            out_specs=pl.BlockSpec((bm, D), row),
            out_shape=jax.ShapeDtypeStruct((N, D), jnp.float32),
        )(*args)
    dw, db = dec
    n_out = dw.shape[1]

    def body_dec(*refs):
        hv = _update_h(k, *refs[:-3])
        refs[-1][...] = (jnp.dot(hv, refs[-3][...],
                                 preferred_element_type=jnp.float32)
                         + refs[-2][...])
    return pl.pallas_call(
        body_dec,
        grid=(N // bm,),
        in_specs=in_specs + [pl.BlockSpec((D, n_out), full),
                             pl.BlockSpec((n_out,), lambda i: (0,))],
        out_specs=pl.BlockSpec((bm, n_out), row),
        out_shape=jax.ShapeDtypeStruct((N, n_out), jnp.float32),
    )(*args, dw, db)


def _decode_body(h_ref, w_ref, b_ref, o_ref):
    o_ref[...] = (jnp.dot(h_ref[...], w_ref[...],
                          preferred_element_type=jnp.float32) + b_ref[...])


def _decode(h, w, b, bm=1000):
    n_out = w.shape[1]
    return pl.pallas_call(
        _decode_body,
        grid=(N // bm,),
        in_specs=[pl.BlockSpec((bm, D), lambda i: (i, 0)),
                  pl.BlockSpec((D, n_out), lambda i: (0, 0)),
                  pl.BlockSpec((1, n_out), lambda i: (0, 0))],
        out_specs=pl.BlockSpec((bm, n_out), lambda i: (i, 0)),
        out_shape=jax.ShapeDtypeStruct((N, n_out), jnp.float32),
    )(h, w, b)


def kernel(x, edge_attr, edge_index, node_W, node_b, edge_W, edge_b,
           conv_edge_W, conv_edge_b, conv_upd_W, conv_upd_b, dec_W, dec_b):
    eidx = edge_index.reshape(2, NCHUNK, CH).transpose(1, 0, 2)
    h = _emb(x, node_W, node_b, bm=1000)
    ea = _emb(edge_attr, edge_W, edge_b, bm=4000)
    sea_all, cnt_all = _sc_edge_accum(ea, eidx)
    nk = conv_edge_W.shape[0]
    for k in range(nk):
        p_all = _sc_spmm(h, eidx)
        dec = (dec_W, dec_b) if k == nk - 1 else None
        h = _update(h, p_all, sea_all, cnt_all, k,
                    conv_edge_W, conv_edge_b, conv_upd_W, conv_upd_b,
                    dec=dec)
    return h
